# Initial kernel scaffold; baseline (speedup 1.0000x reference)
import jax
import jax.numpy as jnp
from jax.experimental import pallas as pl


def kernel(x, y, y_atom_types, W1, b1, W2, b2, gamma, beta):
    raise NotImplementedError("write your pallas kernel here")


# speedup vs baseline: 3.1736x; 3.1736x over previous
import functools

import jax
import jax.numpy as jnp
from jax import lax
from jax.experimental import pallas as pl
from jax.experimental.pallas import tpu as pltpu
from jax.experimental.pallas import tpu_sc as plsc

ATOM_DIMS = 32
N_LAYERS = 3
K = 16
NEG_SLOPE = 0.2
EPS = 1e-5

QB = 128
NCH = 4
QB2 = 256
_MAXI = 2**31 - 1
_NST = 64


def _extract16_full(d_ref, de_ref):
    m_keys = d_ref.shape[1]
    ch = m_keys // NCH

    def chunk_min(c, m):
        sl = pl.ds(c * ch, ch)
        return jnp.minimum(m, jnp.min(d_ref[:, sl], axis=1, keepdims=True))

    m0 = lax.fori_loop(0, NCH, chunk_min,
                       jnp.full((QB, 1), jnp.inf, jnp.float32))

    def extract(j, carry):
        m, idx_acc, dist_acc = carry

        def amin_chunk(c, a):
            sl = pl.ds(c * ch, ch)
            blk = d_ref[:, sl]
            io = lax.broadcasted_iota(jnp.int32, (QB, ch), 1) + c * ch
            t = jnp.where(blk == m, io, _MAXI)
            return jnp.minimum(a, jnp.min(t, axis=1, keepdims=True))

        a = lax.fori_loop(0, NCH, amin_chunk,
                          jnp.full((QB, 1), _MAXI, jnp.int32))

        def mask_chunk(c, carry2):
            nm, ev = carry2
            sl = pl.ds(c * ch, ch)
            blk = d_ref[:, sl]
            io = lax.broadcasted_iota(jnp.int32, (QB, ch), 1) + c * ch
            hit = (blk == m) & (io == a)
            ev = jnp.minimum(ev, jnp.min(
                jnp.where(hit, de_ref[:, sl], jnp.inf),
                axis=1, keepdims=True))
            nblk = jnp.where(hit, jnp.inf, blk)
            d_ref[:, sl] = nblk
            return jnp.minimum(nm, jnp.min(nblk, axis=1, keepdims=True)), ev

        m_next, e = lax.fori_loop(
            0, NCH, mask_chunk,
            (jnp.full((QB, 1), jnp.inf, jnp.float32),
             jnp.full((QB, 1), jnp.inf, jnp.float32)))
        lane = lax.broadcasted_iota(jnp.int32, (QB, K), 1)
        idx_acc = jnp.where(lane == j, a, idx_acc)
        dist_acc = jnp.where(lane == j, e, dist_acc)
        return m_next, idx_acc, dist_acc

    _, idx_acc, dist_acc = lax.fori_loop(
        0, K, extract,
        (m0, jnp.zeros((QB, K), jnp.int32), jnp.zeros((QB, K), jnp.float32)))
    return idx_acc, dist_acc


def _knn_body(x_ref, yt_ref, idx_ref, dist_ref, d_ref, de_ref):
    m_keys = yt_ref.shape[1]
    sw = m_keys // _NST

    x = x_ref[...]
    xb = x.astype(jnp.bfloat16).astype(jnp.float32)
    xsq = jnp.sum(x * x, axis=1, keepdims=True)

    def stripe(t, carry):
        v1, v2, v3, v4, e1, e2, e3, e4, i1, i2, i3, i4 = carry
        sl = pl.ds(t * sw, sw)
        y0 = yt_ref[0:1, sl]
        y1 = yt_ref[1:2, sl]
        y2 = yt_ref[2:3, sl]
        ysq = y0 * y0 + y1 * y1 + y2 * y2
        yb0 = y0.astype(jnp.bfloat16).astype(jnp.float32)
        yb1 = y1.astype(jnp.bfloat16).astype(jnp.float32)
        yb2 = y2.astype(jnp.bfloat16).astype(jnp.float32)
        prod = (xb[:, 0:1] * yb0 + xb[:, 1:2] * yb1 + xb[:, 2:3] * yb2)
        sel = (xsq + ysq) - 2.0 * prod
        d0 = y0 - x[:, 0:1]
        d1 = y1 - x[:, 1:2]
        d2 = y2 - x[:, 2:3]
        ex = d0 * d0 + d1 * d1 + d2 * d2
        d_ref[:, sl] = sel
        de_ref[:, sl] = ex
        gi = lax.broadcasted_iota(jnp.int32, (QB, sw), 1) + t * sw
        c1 = sel < v1
        c2 = sel < v2
        c3 = sel < v3
        c4 = sel < v4
        nv4 = jnp.where(c4, jnp.where(c3, v3, sel), v4)
        ne4 = jnp.where(c4, jnp.where(c3, e3, ex), e4)
        ni4 = jnp.where(c4, jnp.where(c3, i3, gi), i4)
        nv3 = jnp.where(c3, jnp.where(c2, v2, sel), v3)
        ne3 = jnp.where(c3, jnp.where(c2, e2, ex), e3)
        ni3 = jnp.where(c3, jnp.where(c2, i2, gi), i3)
        nv2 = jnp.where(c2, jnp.where(c1, v1, sel), v2)
        ne2 = jnp.where(c2, jnp.where(c1, e1, ex), e2)
        ni2 = jnp.where(c2, jnp.where(c1, i1, gi), i2)
        nv1 = jnp.where(c1, sel, v1)
        ne1 = jnp.where(c1, ex, e1)
        ni1 = jnp.where(c1, gi, i1)
        return (nv1, nv2, nv3, nv4, ne1, ne2, ne3, ne4, ni1, ni2, ni3, ni4)

    inf = jnp.full((QB, sw), jnp.inf, jnp.float32)
    mxi = jnp.full((QB, sw), _MAXI, jnp.int32)
    (v1, v2, v3, v4, e1, e2, e3, e4, i1, i2, i3, i4) = lax.fori_loop(
        0, _NST, stripe,
        (inf, inf, inf, inf, inf, inf, inf, inf, mxi, mxi, mxi, mxi))

    def wextract(j, carry):
        f, fe, fi, cnt, of, idx_acc, dist_acc = carry
        m = jnp.min(f, axis=1, keepdims=True)
        hitf = f == m
        a = jnp.min(jnp.where(hitf, fi, _MAXI), axis=1, keepdims=True)
        win = hitf & (fi == a)
        e = jnp.min(jnp.where(win, fe, jnp.inf), axis=1, keepdims=True)
        lane = lax.broadcasted_iota(jnp.int32, (QB, K), 1)
        idx_acc = jnp.where(lane == j, a, idx_acc)
        dist_acc = jnp.where(lane == j, e, dist_acc)
        nextv = jnp.where(cnt == 1, v2,
                          jnp.where(cnt == 2, v3,
                                    jnp.where(cnt == 3, v4, jnp.inf)))
        nexte = jnp.where(cnt == 1, e2,
                          jnp.where(cnt == 2, e3,
                                    jnp.where(cnt == 3, e4, jnp.inf)))
        nexti = jnp.where(cnt == 1, i2,
                          jnp.where(cnt == 2, i3,
                                    jnp.where(cnt == 3, i4, _MAXI)))
        f = jnp.where(win, nextv, f)
        fe = jnp.where(win, nexte, fe)
        fi = jnp.where(win, nexti, fi)
        of = of | ((j < K - 1) & jnp.any(win & (cnt >= 4)))
        cnt = cnt + win.astype(jnp.int32)
        return f, fe, fi, cnt, of, idx_acc, dist_acc

    ones = jnp.ones((QB, sw), jnp.int32)
    _, _, _, _, overflow, idx_acc, dist_acc = lax.fori_loop(
        0, K, wextract,
        (v1, e1, i1, ones, jnp.bool_(False),
         jnp.zeros((QB, K), jnp.int32), jnp.zeros((QB, K), jnp.float32)))

    idx_acc, dist_acc = lax.cond(
        overflow, lambda _: _extract16_full(d_ref, de_ref),
        lambda _: (idx_acc, dist_acc), None)
    idx_ref[...] = idx_acc
    dist_ref[...] = dist_acc


def _knn(x, y):
    n, _ = x.shape
    m_keys = y.shape[0]
    yt = y.T
    grid = n // QB
    return pl.pallas_call(
        _knn_body,
        grid=(grid,),
        in_specs=[
            pl.BlockSpec((QB, 3), lambda i: (i, 0)),
            pl.BlockSpec((3, m_keys), lambda i: (0, 0)),
        ],
        out_specs=[
            pl.BlockSpec((QB, K), lambda i: (i, 0)),
            pl.BlockSpec((QB, K), lambda i: (i, 0)),
        ],
        out_shape=[
            jax.ShapeDtypeStruct((n, K), jnp.int32),
            jax.ShapeDtypeStruct((n, K), jnp.float32),
        ],
        scratch_shapes=[pltpu.VMEM((QB, m_keys), jnp.float32),
                        pltpu.VMEM((QB, m_keys), jnp.float32)],
    )(x, yt)


def _make_gather(vocab, d, b):
    info = plsc.get_sparse_core_info()
    nc, ns = info.num_cores, info.num_subcores
    nw = nc * ns
    assert b % nw == 0
    b_per_w = b // nw
    chunk = 128
    assert b_per_w % chunk == 0
    mesh = plsc.VectorSubcoreMesh(core_axis_name="c", subcore_axis_name="s")

    @functools.partial(
        pl.kernel,
        mesh=mesh,
        compiler_params=pltpu.CompilerParams(use_tc_tiling_on_sc=False),
        out_type=jax.ShapeDtypeStruct((b, d), jnp.float32),
        scratch_types=[
            pltpu.VMEM((chunk,), jnp.int32),
            pltpu.VMEM((chunk, d), jnp.float32),
            pltpu.SemaphoreType.DMA,
        ],
    )
    def gather_kernel(table_hbm, idx_hbm, out_hbm, idx_v, rows_v, sem):
        wid = lax.axis_index("s") * nc + lax.axis_index("c")
        base = wid * b_per_w

        def body(c, carry):
            off = base + c * chunk
            pltpu.sync_copy(idx_hbm.at[pl.ds(off, chunk)], idx_v)
            pltpu.async_copy(table_hbm.at[idx_v], rows_v, sem).wait()
            pltpu.sync_copy(rows_v, out_hbm.at[pl.ds(off, chunk)])
            return carry

        lax.fori_loop(0, b_per_w // chunk, body, 0)

    return gather_kernel


def _mlp_body(f_ref, w1_ref, b1_ref, w2_ref, b2_ref, gam_ref, bet_ref,
              out_ref):
    h = 2 * ATOM_DIMS + 1
    rows = QB2 * K
    f = f_ref[...]
    pe = jnp.ones((QB2, ATOM_DIMS), jnp.float32)
    hp = jax.lax.Precision.HIGHEST
    for i in range(N_LAYERS):
        w1 = w1_ref[i]
        za = jnp.dot(f, w1[ATOM_DIMS:h, :], precision=hp)
        zb = jnp.dot(pe, w1[:ATOM_DIMS, :], precision=hp)
        z = (za.reshape(QB2, K, h) + zb[:, None, :]
             + b1_ref[i][None, None, :])
        hdn = jnp.where(z >= 0, z, NEG_SLOPE * z)
        msg = jnp.dot(hdn.reshape(rows, h), w2_ref[i], precision=hp)
        msum = (msg.reshape(QB2, K, ATOM_DIMS).sum(axis=1)
                + float(K) * b2_ref[i][None, :])
        halves = []
        for lo in (0, ATOM_DIMS // 2):
            part = msum[:, lo:lo + ATOM_DIMS // 2]
            mu = jnp.mean(part, axis=1, keepdims=True)
            ctr = part - mu
            var = jnp.mean(ctr * ctr, axis=1, keepdims=True)
            halves.append(ctr * lax.rsqrt(var + EPS))
        gn = (jnp.concatenate(halves, axis=1) * gam_ref[i][None, :]
              + bet_ref[i][None, :])
        pe = pe + jnp.where(gn >= 0, gn, NEG_SLOPE * gn)
    out_ref[...] = pe


def _mlp(f, w1, b1, w2, b2, gamma, beta):
    n = f.shape[0] // K
    h = 2 * ATOM_DIMS + 1
    grid = n // QB2
    full = lambda *s: pl.BlockSpec(s, lambda i: (0,) * len(s))
    return pl.pallas_call(
        _mlp_body,
        grid=(grid,),
        in_specs=[
            pl.BlockSpec((QB2 * K, h - ATOM_DIMS * 2 + ATOM_DIMS),
                         lambda i: (i, 0)),
            full(N_LAYERS, h, h),
            full(N_LAYERS, h),
            full(N_LAYERS, h, ATOM_DIMS),
            full(N_LAYERS, ATOM_DIMS),
            full(N_LAYERS, ATOM_DIMS),
            full(N_LAYERS, ATOM_DIMS),
        ],
        out_specs=pl.BlockSpec((QB2, ATOM_DIMS), lambda i: (i, 0)),
        out_shape=jax.ShapeDtypeStruct((n, ATOM_DIMS), jnp.float32),
    )(f, w1, b1, w2, b2, gamma, beta)


def kernel(x, y, y_atom_types, W1, b1, W2, b2, gamma, beta):
    n = x.shape[0]
    vocab, d = y_atom_types.shape
    idx, dists = _knn(x, y)
    gather = _make_gather(vocab, d, n * K)
    g = gather(y_atom_types, idx.reshape(-1))
    f = jnp.concatenate([g, dists.reshape(-1, 1)], axis=1)
    return _mlp(f, W1, b1, W2, b2, gamma, beta)


# speedup vs baseline: 5.3719x; 1.6927x over previous
import functools

import jax
import jax.numpy as jnp
from jax import lax
from jax.experimental import pallas as pl
from jax.experimental.pallas import tpu as pltpu
from jax.experimental.pallas import tpu_sc as plsc

ATOM_DIMS = 32
N_LAYERS = 3
K = 16
NEG_SLOPE = 0.2
EPS = 1e-5

QB = 128
NCH = 4
QB2 = 256
_MAXI = 2**31 - 1
_NST = 64


def _extract16_full(d_ref, de_ref):
    m_keys = d_ref.shape[1]
    ch = m_keys // NCH

    def chunk_min(c, m):
        sl = pl.ds(c * ch, ch)
        return jnp.minimum(m, jnp.min(d_ref[:, sl], axis=1, keepdims=True))

    m0 = lax.fori_loop(0, NCH, chunk_min,
                       jnp.full((QB, 1), jnp.inf, jnp.float32))

    def extract(j, carry):
        m, idx_acc, dist_acc = carry

        def amin_chunk(c, a):
            sl = pl.ds(c * ch, ch)
            blk = d_ref[:, sl]
            io = lax.broadcasted_iota(jnp.int32, (QB, ch), 1) + c * ch
            t = jnp.where(blk == m, io, _MAXI)
            return jnp.minimum(a, jnp.min(t, axis=1, keepdims=True))

        a = lax.fori_loop(0, NCH, amin_chunk,
                          jnp.full((QB, 1), _MAXI, jnp.int32))

        def mask_chunk(c, carry2):
            nm, ev = carry2
            sl = pl.ds(c * ch, ch)
            blk = d_ref[:, sl]
            io = lax.broadcasted_iota(jnp.int32, (QB, ch), 1) + c * ch
            hit = (blk == m) & (io == a)
            ev = jnp.minimum(ev, jnp.min(
                jnp.where(hit, de_ref[:, sl], jnp.inf),
                axis=1, keepdims=True))
            nblk = jnp.where(hit, jnp.inf, blk)
            d_ref[:, sl] = nblk
            return jnp.minimum(nm, jnp.min(nblk, axis=1, keepdims=True)), ev

        m_next, e = lax.fori_loop(
            0, NCH, mask_chunk,
            (jnp.full((QB, 1), jnp.inf, jnp.float32),
             jnp.full((QB, 1), jnp.inf, jnp.float32)))
        lane = lax.broadcasted_iota(jnp.int32, (QB, K), 1)
        idx_acc = jnp.where(lane == j, a, idx_acc)
        dist_acc = jnp.where(lane == j, e, dist_acc)
        return m_next, idx_acc, dist_acc

    _, idx_acc, dist_acc = lax.fori_loop(
        0, K, extract,
        (m0, jnp.zeros((QB, K), jnp.int32), jnp.zeros((QB, K), jnp.float32)))
    return idx_acc, dist_acc


def _knn_body(x_ref, yt_ref, idx_ref, dist_ref, d_ref, de_ref):
    m_keys = yt_ref.shape[1]
    sw = m_keys // _NST

    x = x_ref[...]
    xb = x.astype(jnp.bfloat16).astype(jnp.float32)
    xsq = jnp.sum(x * x, axis=1, keepdims=True)

    def stripe(t, carry):
        v1, v2, v3, v4, e1, e2, e3, e4, i1, i2, i3, i4 = carry
        sl = pl.ds(t * sw, sw)
        y0 = yt_ref[0:1, sl]
        y1 = yt_ref[1:2, sl]
        y2 = yt_ref[2:3, sl]
        ysq = y0 * y0 + y1 * y1 + y2 * y2
        yb0 = y0.astype(jnp.bfloat16).astype(jnp.float32)
        yb1 = y1.astype(jnp.bfloat16).astype(jnp.float32)
        yb2 = y2.astype(jnp.bfloat16).astype(jnp.float32)
        prod = (xb[:, 0:1] * yb0 + xb[:, 1:2] * yb1 + xb[:, 2:3] * yb2)
        sel = (xsq + ysq) - 2.0 * prod
        d0 = y0 - x[:, 0:1]
        d1 = y1 - x[:, 1:2]
        d2 = y2 - x[:, 2:3]
        ex = d0 * d0 + d1 * d1 + d2 * d2
        d_ref[:, sl] = sel
        de_ref[:, sl] = ex
        gi = lax.broadcasted_iota(jnp.int32, (QB, sw), 1) + t * sw
        c1 = sel < v1
        c2 = sel < v2
        c3 = sel < v3
        c4 = sel < v4
        nv4 = jnp.where(c4, jnp.where(c3, v3, sel), v4)
        ne4 = jnp.where(c4, jnp.where(c3, e3, ex), e4)
        ni4 = jnp.where(c4, jnp.where(c3, i3, gi), i4)
        nv3 = jnp.where(c3, jnp.where(c2, v2, sel), v3)
        ne3 = jnp.where(c3, jnp.where(c2, e2, ex), e3)
        ni3 = jnp.where(c3, jnp.where(c2, i2, gi), i3)
        nv2 = jnp.where(c2, jnp.where(c1, v1, sel), v2)
        ne2 = jnp.where(c2, jnp.where(c1, e1, ex), e2)
        ni2 = jnp.where(c2, jnp.where(c1, i1, gi), i2)
        nv1 = jnp.where(c1, sel, v1)
        ne1 = jnp.where(c1, ex, e1)
        ni1 = jnp.where(c1, gi, i1)
        return (nv1, nv2, nv3, nv4, ne1, ne2, ne3, ne4, ni1, ni2, ni3, ni4)

    inf = jnp.full((QB, sw), jnp.inf, jnp.float32)
    mxi = jnp.full((QB, sw), _MAXI, jnp.int32)
    (v1, v2, v3, v4, e1, e2, e3, e4, i1, i2, i3, i4) = lax.fori_loop(
        0, _NST, stripe,
        (inf, inf, inf, inf, inf, inf, inf, inf, mxi, mxi, mxi, mxi))

    def wextract(j, carry):
        f, fe, fi, cnt, of, idx_acc, dist_acc = carry
        m = jnp.min(f, axis=1, keepdims=True)
        hitf = f == m
        a = jnp.min(jnp.where(hitf, fi, _MAXI), axis=1, keepdims=True)
        win = hitf & (fi == a)
        e = jnp.min(jnp.where(win, fe, jnp.inf), axis=1, keepdims=True)
        lane = lax.broadcasted_iota(jnp.int32, (QB, K), 1)
        idx_acc = jnp.where(lane == j, a, idx_acc)
        dist_acc = jnp.where(lane == j, e, dist_acc)
        nextv = jnp.where(cnt == 1, v2,
                          jnp.where(cnt == 2, v3,
                                    jnp.where(cnt == 3, v4, jnp.inf)))
        nexte = jnp.where(cnt == 1, e2,
                          jnp.where(cnt == 2, e3,
                                    jnp.where(cnt == 3, e4, jnp.inf)))
        nexti = jnp.where(cnt == 1, i2,
                          jnp.where(cnt == 2, i3,
                                    jnp.where(cnt == 3, i4, _MAXI)))
        f = jnp.where(win, nextv, f)
        fe = jnp.where(win, nexte, fe)
        fi = jnp.where(win, nexti, fi)
        of = of | ((j < K - 1) & jnp.any(win & (cnt >= 4)))
        cnt = cnt + win.astype(jnp.int32)
        return f, fe, fi, cnt, of, idx_acc, dist_acc

    ones = jnp.ones((QB, sw), jnp.int32)
    _, _, _, _, overflow, idx_acc, dist_acc = lax.fori_loop(
        0, K, wextract,
        (v1, e1, i1, ones, jnp.bool_(False),
         jnp.zeros((QB, K), jnp.int32), jnp.zeros((QB, K), jnp.float32)))

    idx_acc, dist_acc = lax.cond(
        overflow, lambda _: _extract16_full(d_ref, de_ref),
        lambda _: (idx_acc, dist_acc), None)
    idx_ref[...] = idx_acc
    dist_ref[...] = dist_acc


def _knn(x, y):
    n, _ = x.shape
    m_keys = y.shape[0]
    yt = y.T
    grid = n // QB
    return pl.pallas_call(
        _knn_body,
        grid=(grid,),
        in_specs=[
            pl.BlockSpec((QB, 3), lambda i: (i, 0)),
            pl.BlockSpec((3, m_keys), lambda i: (0, 0)),
        ],
        out_specs=[
            pl.BlockSpec((QB, K), lambda i: (i, 0)),
            pl.BlockSpec((QB, K), lambda i: (i, 0)),
        ],
        out_shape=[
            jax.ShapeDtypeStruct((n, K), jnp.int32),
            jax.ShapeDtypeStruct((n, K), jnp.float32),
        ],
        scratch_shapes=[pltpu.VMEM((QB, m_keys), jnp.float32),
                        pltpu.VMEM((QB, m_keys), jnp.float32)],
    )(x, yt)


def _make_gather(vocab, d, b):
    info = plsc.get_sparse_core_info()
    nc, ns = info.num_cores, info.num_subcores
    nw = nc * ns
    assert b % nw == 0
    b_per_w = b // nw
    chunk = 128
    assert b_per_w % chunk == 0
    mesh = plsc.VectorSubcoreMesh(core_axis_name="c", subcore_axis_name="s")

    @functools.partial(
        pl.kernel,
        mesh=mesh,
        compiler_params=pltpu.CompilerParams(use_tc_tiling_on_sc=False),
        out_type=jax.ShapeDtypeStruct((b, d), jnp.float32),
        scratch_types=[
            pltpu.VMEM((chunk,), jnp.int32),
            pltpu.VMEM((chunk, d), jnp.float32),
            pltpu.SemaphoreType.DMA,
        ],
    )
    def gather_kernel(table_hbm, idx_hbm, out_hbm, idx_v, rows_v, sem):
        wid = lax.axis_index("s") * nc + lax.axis_index("c")
        base = wid * b_per_w

        def body(c, carry):
            off = base + c * chunk
            pltpu.sync_copy(idx_hbm.at[pl.ds(off, chunk)], idx_v)
            pltpu.async_copy(table_hbm.at[idx_v], rows_v, sem).wait()
            pltpu.sync_copy(rows_v, out_hbm.at[pl.ds(off, chunk)])
            return carry

        lax.fori_loop(0, b_per_w // chunk, body, 0)

    return gather_kernel


def _mlp_body(f_ref, w1_ref, b1_ref, w2_ref, b2_ref, gam_ref, bet_ref,
              out_ref):
    h = 2 * ATOM_DIMS + 1
    rows = QB2 * K
    f = f_ref[...]
    pe = jnp.ones((QB2, ATOM_DIMS), jnp.float32)
    hp = jax.lax.Precision.HIGHEST
    for i in range(N_LAYERS):
        w1 = w1_ref[i]
        za = jnp.dot(f, w1[ATOM_DIMS:h, :], precision=hp)
        zb = jnp.dot(pe, w1[:ATOM_DIMS, :], precision=hp)
        z = (za.reshape(QB2, K, h) + zb[:, None, :]
             + b1_ref[i][None, None, :])
        hdn = jnp.where(z >= 0, z, NEG_SLOPE * z)
        msg = jnp.dot(hdn.reshape(rows, h), w2_ref[i], precision=hp)
        msum = (msg.reshape(QB2, K, ATOM_DIMS).sum(axis=1)
                + float(K) * b2_ref[i][None, :])
        halves = []
        for lo in (0, ATOM_DIMS // 2):
            part = msum[:, lo:lo + ATOM_DIMS // 2]
            mu = jnp.mean(part, axis=1, keepdims=True)
            ctr = part - mu
            var = jnp.mean(ctr * ctr, axis=1, keepdims=True)
            halves.append(ctr * lax.rsqrt(var + EPS))
        gn = (jnp.concatenate(halves, axis=1) * gam_ref[i][None, :]
              + bet_ref[i][None, :])
        pe = pe + jnp.where(gn >= 0, gn, NEG_SLOPE * gn)
    out_ref[...] = pe


def _mlp(f, w1, b1, w2, b2, gamma, beta):
    n = f.shape[0] // K
    h = 2 * ATOM_DIMS + 1
    grid = n // QB2
    full = lambda *s: pl.BlockSpec(s, lambda i: (0,) * len(s))
    return pl.pallas_call(
        _mlp_body,
        grid=(grid,),
        in_specs=[
            pl.BlockSpec((QB2 * K, h - ATOM_DIMS * 2 + ATOM_DIMS),
                         lambda i: (i, 0)),
            full(N_LAYERS, h, h),
            full(N_LAYERS, h),
            full(N_LAYERS, h, ATOM_DIMS),
            full(N_LAYERS, ATOM_DIMS),
            full(N_LAYERS, ATOM_DIMS),
            full(N_LAYERS, ATOM_DIMS),
        ],
        out_specs=pl.BlockSpec((QB2, ATOM_DIMS), lambda i: (i, 0)),
        out_shape=jax.ShapeDtypeStruct((n, ATOM_DIMS), jnp.float32),
    )(f, w1, b1, w2, b2, gamma, beta)


def _run(x, y, y_atom_types, W1, b1, W2, b2, gamma, beta):
    n = x.shape[0]
    vocab, d = y_atom_types.shape
    idx, dists = _knn(x, y)
    gather = _make_gather(vocab, d, n * K)
    g = gather(y_atom_types, idx.reshape(-1))
    f = jnp.concatenate([g, dists.reshape(-1, 1)], axis=1)
    return _mlp(f, W1, b1, W2, b2, gamma, beta)


def kernel(x, y, y_atom_types, W1, b1, W2, b2, gamma, beta):
    devs = jax.devices()
    nd = len(devs)
    n = x.shape[0]
    if nd > 1 and n % (nd * QB2 * 4) == 0:
        mesh = jax.make_mesh((nd,), ("q",))
        p = jax.sharding.PartitionSpec
        ns = lambda spec: jax.sharding.NamedSharding(mesh, spec)
        rep = p()
        args = [jax.reshard(a, ns(s)) for a, s in zip(
            (x, y, y_atom_types, W1, b1, W2, b2, gamma, beta),
            (p("q"), rep, rep, rep, rep, rep, rep, rep, rep))]
        fn = jax.shard_map(
            _run, mesh=mesh,
            in_specs=(p("q"), rep, rep, rep, rep, rep, rep, rep, rep),
            out_specs=p("q"), check_vma=False)
        return fn(*args)
    return _run(x, y, y_atom_types, W1, b1, W2, b2, gamma, beta)


# speedup vs baseline: 6.3924x; 1.1900x over previous
import functools

import jax
import jax.numpy as jnp
from jax import lax
from jax.experimental import pallas as pl
from jax.experimental.pallas import tpu as pltpu
from jax.experimental.pallas import tpu_sc as plsc

ATOM_DIMS = 32
N_LAYERS = 3
K = 16
NEG_SLOPE = 0.2
EPS = 1e-5

QB = 128
NCH = 4
QB2 = 256
_MAXI = 2**31 - 1
_NST = 64


def _extract16_full(d_ref):
    m_keys = d_ref.shape[1]
    ch = m_keys // NCH

    def chunk_min(c, m):
        sl = pl.ds(c * ch, ch)
        return jnp.minimum(m, jnp.min(d_ref[:, sl], axis=1, keepdims=True))

    m0 = lax.fori_loop(0, NCH, chunk_min,
                       jnp.full((QB, 1), jnp.inf, jnp.float32))

    def extract(j, carry):
        m, idx_acc = carry

        def amin_chunk(c, a):
            sl = pl.ds(c * ch, ch)
            blk = d_ref[:, sl]
            io = lax.broadcasted_iota(jnp.int32, (QB, ch), 1) + c * ch
            t = jnp.where(blk == m, io, _MAXI)
            return jnp.minimum(a, jnp.min(t, axis=1, keepdims=True))

        a = lax.fori_loop(0, NCH, amin_chunk,
                          jnp.full((QB, 1), _MAXI, jnp.int32))

        def mask_chunk(c, nm):
            sl = pl.ds(c * ch, ch)
            blk = d_ref[:, sl]
            io = lax.broadcasted_iota(jnp.int32, (QB, ch), 1) + c * ch
            hit = (blk == m) & (io == a)
            nblk = jnp.where(hit, jnp.inf, blk)
            d_ref[:, sl] = nblk
            return jnp.minimum(nm, jnp.min(nblk, axis=1, keepdims=True))

        m_next = lax.fori_loop(0, NCH, mask_chunk,
                               jnp.full((QB, 1), jnp.inf, jnp.float32))
        lane = lax.broadcasted_iota(jnp.int32, (QB, K), 1)
        idx_acc = jnp.where(lane == j, a, idx_acc)
        return m_next, idx_acc

    _, idx_acc = lax.fori_loop(
        0, K, extract, (m0, jnp.zeros((QB, K), jnp.int32)))
    return idx_acc


def _knn_body(x_ref, yt_ref, idx_ref, d_ref):
    m_keys = yt_ref.shape[1]
    sw = m_keys // _NST

    x = x_ref[...]
    xb = x.astype(jnp.bfloat16).astype(jnp.float32)
    xsq = jnp.sum(x * x, axis=1, keepdims=True)

    def stripe(t, carry):
        v1, v2, v3, v4, i1, i2, i3, i4 = carry
        sl = pl.ds(t * sw, sw)
        y0 = yt_ref[0:1, sl]
        y1 = yt_ref[1:2, sl]
        y2 = yt_ref[2:3, sl]
        ysq = y0 * y0 + y1 * y1 + y2 * y2
        yb0 = y0.astype(jnp.bfloat16).astype(jnp.float32)
        yb1 = y1.astype(jnp.bfloat16).astype(jnp.float32)
        yb2 = y2.astype(jnp.bfloat16).astype(jnp.float32)
        prod = (xb[:, 0:1] * yb0 + xb[:, 1:2] * yb1 + xb[:, 2:3] * yb2)
        sel = (xsq + ysq) - 2.0 * prod
        d_ref[:, sl] = sel
        gi = lax.broadcasted_iota(jnp.int32, (QB, sw), 1) + t * sw
        c1 = sel < v1
        c2 = sel < v2
        c3 = sel < v3
        c4 = sel < v4
        nv4 = jnp.where(c4, jnp.where(c3, v3, sel), v4)
        ni4 = jnp.where(c4, jnp.where(c3, i3, gi), i4)
        nv3 = jnp.where(c3, jnp.where(c2, v2, sel), v3)
        ni3 = jnp.where(c3, jnp.where(c2, i2, gi), i3)
        nv2 = jnp.where(c2, jnp.where(c1, v1, sel), v2)
        ni2 = jnp.where(c2, jnp.where(c1, i1, gi), i2)
        nv1 = jnp.where(c1, sel, v1)
        ni1 = jnp.where(c1, gi, i1)
        return (nv1, nv2, nv3, nv4, ni1, ni2, ni3, ni4)

    inf = jnp.full((QB, sw), jnp.inf, jnp.float32)
    mxi = jnp.full((QB, sw), _MAXI, jnp.int32)
    (v1, v2, v3, v4, i1, i2, i3, i4) = lax.fori_loop(
        0, _NST, stripe, (inf, inf, inf, inf, mxi, mxi, mxi, mxi))

    def wextract(j, carry):
        f, fi, cnt, of, idx_acc = carry
        m = jnp.min(f, axis=1, keepdims=True)
        hitf = f == m
        a = jnp.min(jnp.where(hitf, fi, _MAXI), axis=1, keepdims=True)
        win = hitf & (fi == a)
        lane = lax.broadcasted_iota(jnp.int32, (QB, K), 1)
        idx_acc = jnp.where(lane == j, a, idx_acc)
        nextv = jnp.where(cnt == 1, v2,
                          jnp.where(cnt == 2, v3,
                                    jnp.where(cnt == 3, v4, jnp.inf)))
        nexti = jnp.where(cnt == 1, i2,
                          jnp.where(cnt == 2, i3,
                                    jnp.where(cnt == 3, i4, _MAXI)))
        f = jnp.where(win, nextv, f)
        fi = jnp.where(win, nexti, fi)
        of = of | ((j < K - 1) & jnp.any(win & (cnt >= 4)))
        cnt = cnt + win.astype(jnp.int32)
        return f, fi, cnt, of, idx_acc

    _, _, _, overflow, idx_acc = lax.fori_loop(
        0, K, wextract,
        (v1, i1, jnp.ones((QB, sw), jnp.int32), jnp.bool_(False),
         jnp.zeros((QB, K), jnp.int32)))

    idx_acc = lax.cond(overflow, lambda _: _extract16_full(d_ref),
                       lambda _: idx_acc, None)
    idx_ref[...] = idx_acc


def _knn(x, y):
    n, _ = x.shape
    m_keys = y.shape[0]
    yt = y.T
    grid = n // QB
    return pl.pallas_call(
        _knn_body,
        grid=(grid,),
        in_specs=[
            pl.BlockSpec((QB, 3), lambda i: (i, 0)),
            pl.BlockSpec((3, m_keys), lambda i: (0, 0)),
        ],
        out_specs=pl.BlockSpec((QB, K), lambda i: (i, 0)),
        out_shape=jax.ShapeDtypeStruct((n, K), jnp.int32),
        scratch_shapes=[pltpu.VMEM((QB, m_keys), jnp.float32)],
    )(x, yt)


def _make_gather(vocab, d, b):
    info = plsc.get_sparse_core_info()
    nc, ns = info.num_cores, info.num_subcores
    nw = nc * ns
    assert b % nw == 0
    b_per_w = b // nw
    chunk = 128
    assert b_per_w % chunk == 0
    mesh = plsc.VectorSubcoreMesh(core_axis_name="c", subcore_axis_name="s")

    @functools.partial(
        pl.kernel,
        mesh=mesh,
        compiler_params=pltpu.CompilerParams(use_tc_tiling_on_sc=False),
        out_type=[jax.ShapeDtypeStruct((b, d), jnp.float32),
                  jax.ShapeDtypeStruct((b, 16), jnp.float32)],
        scratch_types=[
            pltpu.VMEM((chunk,), jnp.int32),
            pltpu.VMEM((chunk, d), jnp.float32),
            pltpu.VMEM((chunk, 16), jnp.float32),
            pltpu.SemaphoreType.DMA,
        ],
    )
    def gather_kernel(table_hbm, idx_hbm, y16_hbm, out_hbm,
                      yg_hbm, idx_v, rows_v, yrow_v, sem):
        wid = lax.axis_index("s") * nc + lax.axis_index("c")
        base = wid * b_per_w

        def body(c, carry):
            off = base + c * chunk
            pltpu.sync_copy(idx_hbm.at[pl.ds(off, chunk)], idx_v)
            pltpu.async_copy(table_hbm.at[idx_v], rows_v, sem).wait()
            pltpu.sync_copy(rows_v, out_hbm.at[pl.ds(off, chunk)])
            pltpu.async_copy(y16_hbm.at[idx_v], yrow_v, sem).wait()
            pltpu.sync_copy(yrow_v, yg_hbm.at[pl.ds(off, chunk)])
            return carry

        lax.fori_loop(0, b_per_w // chunk, body, 0)

    return gather_kernel


def _mlp_body(g_ref, yg_ref, x_ref, w1_ref, b1_ref, w2_ref, b2_ref,
              gam_ref, bet_ref, out_ref):
    h = 2 * ATOM_DIMS + 1
    rows = QB2 * K
    xe = jnp.broadcast_to(x_ref[...][:, None, :], (QB2, K, 16))
    df = yg_ref[...] - xe.reshape(rows, 16)
    dist = jnp.sum(df * df, axis=1, keepdims=True)
    f = jnp.concatenate([g_ref[...], dist], axis=1)
    pe = jnp.ones((QB2, ATOM_DIMS), jnp.float32)
    hp = jax.lax.Precision.HIGHEST
    for i in range(N_LAYERS):
        w1 = w1_ref[i]
        za = jnp.dot(f, w1[ATOM_DIMS:h, :], precision=hp)
        zb = jnp.dot(pe, w1[:ATOM_DIMS, :], precision=hp)
        z = (za.reshape(QB2, K, h) + zb[:, None, :]
             + b1_ref[i][None, None, :])
        hdn = jnp.where(z >= 0, z, NEG_SLOPE * z)
        msg = jnp.dot(hdn.reshape(rows, h), w2_ref[i], precision=hp)
        msum = (msg.reshape(QB2, K, ATOM_DIMS).sum(axis=1)
                + float(K) * b2_ref[i][None, :])
        halves = []
        for lo in (0, ATOM_DIMS // 2):
            part = msum[:, lo:lo + ATOM_DIMS // 2]
            mu = jnp.mean(part, axis=1, keepdims=True)
            ctr = part - mu
            var = jnp.mean(ctr * ctr, axis=1, keepdims=True)
            halves.append(ctr * lax.rsqrt(var + EPS))
        gn = (jnp.concatenate(halves, axis=1) * gam_ref[i][None, :]
              + bet_ref[i][None, :])
        pe = pe + jnp.where(gn >= 0, gn, NEG_SLOPE * gn)
    out_ref[...] = pe


def _mlp(g, ygath, x16, w1, b1, w2, b2, gamma, beta):
    n = g.shape[0] // K
    h = 2 * ATOM_DIMS + 1
    grid = n // QB2
    full = lambda *s: pl.BlockSpec(s, lambda i: (0,) * len(s))
    return pl.pallas_call(
        _mlp_body,
        grid=(grid,),
        in_specs=[
            pl.BlockSpec((QB2 * K, ATOM_DIMS), lambda i: (i, 0)),
            pl.BlockSpec((QB2 * K, 16), lambda i: (i, 0)),
            pl.BlockSpec((QB2, 16), lambda i: (i, 0)),
            full(N_LAYERS, h, h),
            full(N_LAYERS, h),
            full(N_LAYERS, h, ATOM_DIMS),
            full(N_LAYERS, ATOM_DIMS),
            full(N_LAYERS, ATOM_DIMS),
            full(N_LAYERS, ATOM_DIMS),
        ],
        out_specs=pl.BlockSpec((QB2, ATOM_DIMS), lambda i: (i, 0)),
        out_shape=jax.ShapeDtypeStruct((n, ATOM_DIMS), jnp.float32),
    )(g, ygath, x16, w1, b1, w2, b2, gamma, beta)


def _run(x, y, y_atom_types, W1, b1, W2, b2, gamma, beta):
    n = x.shape[0]
    vocab, d = y_atom_types.shape
    idx = _knn(x, y)
    y16 = jnp.pad(y, ((0, 0), (0, 16 - y.shape[1])))
    x16 = jnp.pad(x, ((0, 0), (0, 16 - x.shape[1])))
    gather = _make_gather(vocab, d, n * K)
    g, ygath = gather(y_atom_types, idx.reshape(-1), y16)
    return _mlp(g, ygath, x16, W1, b1, W2, b2, gamma, beta)


def kernel(x, y, y_atom_types, W1, b1, W2, b2, gamma, beta):
    devs = jax.devices()
    nd = len(devs)
    n = x.shape[0]
    if nd > 1 and n % (nd * QB2 * 4) == 0:
        mesh = jax.make_mesh((nd,), ("q",))
        p = jax.sharding.PartitionSpec
        ns = lambda spec: jax.sharding.NamedSharding(mesh, spec)
        rep = p()
        args = [jax.reshard(a, ns(s)) for a, s in zip(
            (x, y, y_atom_types, W1, b1, W2, b2, gamma, beta),
            (p("q"), rep, rep, rep, rep, rep, rep, rep, rep))]
        fn = jax.shard_map(
            _run, mesh=mesh,
            in_specs=(p("q"), rep, rep, rep, rep, rep, rep, rep, rep),
            out_specs=p("q"), check_vma=False)
        return fn(*args)
    return _run(x, y, y_atom_types, W1, b1, W2, b2, gamma, beta)


# speedup vs baseline: 6.9980x; 1.0947x over previous
import functools

import jax
import jax.numpy as jnp
from jax import lax
from jax.experimental import pallas as pl
from jax.experimental.pallas import tpu as pltpu
from jax.experimental.pallas import tpu_sc as plsc

ATOM_DIMS = 32
N_LAYERS = 3
K = 16
NEG_SLOPE = 0.2
EPS = 1e-5

QB = 128
NCH = 4
QB2 = 256
_MAXI = 2**31 - 1
_NST = 64


def _extract16_full(d_ref):
    m_keys = d_ref.shape[1]
    ch = m_keys // NCH

    def chunk_min(c, m):
        sl = pl.ds(c * ch, ch)
        return jnp.minimum(m, jnp.min(d_ref[:, sl], axis=1, keepdims=True))

    m0 = lax.fori_loop(0, NCH, chunk_min,
                       jnp.full((QB, 1), jnp.inf, jnp.float32))

    def extract(j, carry):
        m, idx_acc = carry

        def amin_chunk(c, a):
            sl = pl.ds(c * ch, ch)
            blk = d_ref[:, sl]
            io = lax.broadcasted_iota(jnp.int32, (QB, ch), 1) + c * ch
            t = jnp.where(blk == m, io, _MAXI)
            return jnp.minimum(a, jnp.min(t, axis=1, keepdims=True))

        a = lax.fori_loop(0, NCH, amin_chunk,
                          jnp.full((QB, 1), _MAXI, jnp.int32))

        def mask_chunk(c, nm):
            sl = pl.ds(c * ch, ch)
            blk = d_ref[:, sl]
            io = lax.broadcasted_iota(jnp.int32, (QB, ch), 1) + c * ch
            hit = (blk == m) & (io == a)
            nblk = jnp.where(hit, jnp.inf, blk)
            d_ref[:, sl] = nblk
            return jnp.minimum(nm, jnp.min(nblk, axis=1, keepdims=True))

        m_next = lax.fori_loop(0, NCH, mask_chunk,
                               jnp.full((QB, 1), jnp.inf, jnp.float32))
        lane = lax.broadcasted_iota(jnp.int32, (QB, K), 1)
        idx_acc = jnp.where(lane == j, a, idx_acc)
        return m_next, idx_acc

    _, idx_acc = lax.fori_loop(
        0, K, extract, (m0, jnp.zeros((QB, K), jnp.int32)))
    return idx_acc


def _knn_body(x_ref, yt_ref, idx_ref, d_ref):
    m_keys = yt_ref.shape[1]
    sw = m_keys // _NST

    x = x_ref[...]
    xb = x.astype(jnp.bfloat16).astype(jnp.float32)
    xsq = jnp.sum(x * x, axis=1, keepdims=True)

    def stripe(t, carry):
        v1, v2, v3, v4, i1, i2, i3, i4 = carry
        sl = pl.ds(t * sw, sw)
        y0 = yt_ref[0:1, sl]
        y1 = yt_ref[1:2, sl]
        y2 = yt_ref[2:3, sl]
        ysq = y0 * y0 + y1 * y1 + y2 * y2
        yb0 = y0.astype(jnp.bfloat16).astype(jnp.float32)
        yb1 = y1.astype(jnp.bfloat16).astype(jnp.float32)
        yb2 = y2.astype(jnp.bfloat16).astype(jnp.float32)
        prod = (xb[:, 0:1] * yb0 + xb[:, 1:2] * yb1 + xb[:, 2:3] * yb2)
        sel = (xsq + ysq) - 2.0 * prod
        d_ref[:, sl] = sel
        gi = lax.broadcasted_iota(jnp.int32, (QB, sw), 1) + t * sw
        c1 = sel < v1
        c2 = sel < v2
        c3 = sel < v3
        c4 = sel < v4
        nv4 = jnp.where(c4, jnp.where(c3, v3, sel), v4)
        ni4 = jnp.where(c4, jnp.where(c3, i3, gi), i4)
        nv3 = jnp.where(c3, jnp.where(c2, v2, sel), v3)
        ni3 = jnp.where(c3, jnp.where(c2, i2, gi), i3)
        nv2 = jnp.where(c2, jnp.where(c1, v1, sel), v2)
        ni2 = jnp.where(c2, jnp.where(c1, i1, gi), i2)
        nv1 = jnp.where(c1, sel, v1)
        ni1 = jnp.where(c1, gi, i1)
        return (nv1, nv2, nv3, nv4, ni1, ni2, ni3, ni4)

    inf = jnp.full((QB, sw), jnp.inf, jnp.float32)
    mxi = jnp.full((QB, sw), _MAXI, jnp.int32)
    (v1, v2, v3, v4, i1, i2, i3, i4) = lax.fori_loop(
        0, _NST, stripe, (inf, inf, inf, inf, mxi, mxi, mxi, mxi))

    def wextract(j, carry):
        f, fi, cnt, of, idx_acc = carry
        m = jnp.min(f, axis=1, keepdims=True)
        hitf = f == m
        a = jnp.min(jnp.where(hitf, fi, _MAXI), axis=1, keepdims=True)
        win = hitf & (fi == a)
        lane = lax.broadcasted_iota(jnp.int32, (QB, K), 1)
        idx_acc = jnp.where(lane == j, a, idx_acc)
        nextv = jnp.where(cnt == 1, v2,
                          jnp.where(cnt == 2, v3,
                                    jnp.where(cnt == 3, v4, jnp.inf)))
        nexti = jnp.where(cnt == 1, i2,
                          jnp.where(cnt == 2, i3,
                                    jnp.where(cnt == 3, i4, _MAXI)))
        f = jnp.where(win, nextv, f)
        fi = jnp.where(win, nexti, fi)
        of = of | ((j < K - 1) & jnp.any(win & (cnt >= 4)))
        cnt = cnt + win.astype(jnp.int32)
        return f, fi, cnt, of, idx_acc

    _, _, _, overflow, idx_acc = lax.fori_loop(
        0, K, wextract,
        (v1, i1, jnp.ones((QB, sw), jnp.int32), jnp.bool_(False),
         jnp.zeros((QB, K), jnp.int32)))

    idx_acc = lax.cond(overflow, lambda _: _extract16_full(d_ref),
                       lambda _: idx_acc, None)
    idx_ref[...] = idx_acc


def _knn(x, y):
    n, _ = x.shape
    m_keys = y.shape[0]
    yt = y.T
    grid = n // QB
    return pl.pallas_call(
        _knn_body,
        grid=(grid,),
        in_specs=[
            pl.BlockSpec((QB, 3), lambda i: (i, 0)),
            pl.BlockSpec((3, m_keys), lambda i: (0, 0)),
        ],
        out_specs=pl.BlockSpec((QB, K), lambda i: (i, 0)),
        out_shape=jax.ShapeDtypeStruct((n, K), jnp.int32),
        scratch_shapes=[pltpu.VMEM((QB, m_keys), jnp.float32)],
    )(x, yt)


def _make_gather(vocab, d, b):
    info = plsc.get_sparse_core_info()
    nc, ns = info.num_cores, info.num_subcores
    nw = nc * ns
    assert b % nw == 0
    b_per_w = b // nw
    chunk = 128
    assert b_per_w % chunk == 0
    mesh = plsc.VectorSubcoreMesh(core_axis_name="c", subcore_axis_name="s")

    @functools.partial(
        pl.kernel,
        mesh=mesh,
        compiler_params=pltpu.CompilerParams(use_tc_tiling_on_sc=False),
        out_type=[jax.ShapeDtypeStruct((b, d), jnp.float32),
                  jax.ShapeDtypeStruct((b, 16), jnp.float32)],
        scratch_types=[
            pltpu.VMEM((chunk,), jnp.int32),
            pltpu.VMEM((chunk, d), jnp.float32),
            pltpu.VMEM((chunk, 16), jnp.float32),
            pltpu.SemaphoreType.DMA,
        ],
    )
    def gather_kernel(table_hbm, idx_hbm, y16_hbm, out_hbm,
                      yg_hbm, idx_v, rows_v, yrow_v, sem):
        wid = lax.axis_index("s") * nc + lax.axis_index("c")
        base = wid * b_per_w

        def body(c, carry):
            off = base + c * chunk
            pltpu.sync_copy(idx_hbm.at[pl.ds(off, chunk)], idx_v)
            pltpu.async_copy(table_hbm.at[idx_v], rows_v, sem).wait()
            pltpu.sync_copy(rows_v, out_hbm.at[pl.ds(off, chunk)])
            pltpu.async_copy(y16_hbm.at[idx_v], yrow_v, sem).wait()
            pltpu.sync_copy(yrow_v, yg_hbm.at[pl.ds(off, chunk)])
            return carry

        lax.fori_loop(0, b_per_w // chunk, body, 0)

    return gather_kernel


def _mlp_body(g_ref, yg_ref, x_ref, w1_ref, b1_ref, w2_ref, b2_ref,
              gam_ref, bet_ref, out_ref):
    h = 2 * ATOM_DIMS + 1
    rows = QB2 * K
    xe = jnp.broadcast_to(x_ref[...][:, None, :], (QB2, K, 16))
    df = yg_ref[...] - xe.reshape(rows, 16)
    dist = jnp.sum(df * df, axis=1, keepdims=True)
    f = jnp.concatenate([g_ref[...], dist], axis=1)
    pe = jnp.ones((QB2, ATOM_DIMS), jnp.float32)

    def dot3(a, b):
        ah = a.astype(jnp.bfloat16)
        al = (a - ah.astype(jnp.float32)).astype(jnp.bfloat16)
        bh = b.astype(jnp.bfloat16)
        bl = (b - bh.astype(jnp.float32)).astype(jnp.bfloat16)
        dp = lambda u, v: jax.lax.dot(u, v,
                                      preferred_element_type=jnp.float32)
        return dp(ah, bh) + dp(ah, bl) + dp(al, bh)

    for i in range(N_LAYERS):
        w1 = w1_ref[i]
        za = dot3(f, w1[ATOM_DIMS:h, :])
        zb = dot3(pe, w1[:ATOM_DIMS, :])
        z = (za.reshape(QB2, K, h) + zb[:, None, :]
             + b1_ref[i][None, None, :])
        hdn = jnp.where(z >= 0, z, NEG_SLOPE * z)
        msg = dot3(hdn.reshape(rows, h), w2_ref[i])
        msum = (msg.reshape(QB2, K, ATOM_DIMS).sum(axis=1)
                + float(K) * b2_ref[i][None, :])
        halves = []
        for lo in (0, ATOM_DIMS // 2):
            part = msum[:, lo:lo + ATOM_DIMS // 2]
            mu = jnp.mean(part, axis=1, keepdims=True)
            ctr = part - mu
            var = jnp.mean(ctr * ctr, axis=1, keepdims=True)
            halves.append(ctr * lax.rsqrt(var + EPS))
        gn = (jnp.concatenate(halves, axis=1) * gam_ref[i][None, :]
              + bet_ref[i][None, :])
        pe = pe + jnp.where(gn >= 0, gn, NEG_SLOPE * gn)
    out_ref[...] = pe


def _mlp(g, ygath, x16, w1, b1, w2, b2, gamma, beta):
    n = g.shape[0] // K
    h = 2 * ATOM_DIMS + 1
    grid = n // QB2
    full = lambda *s: pl.BlockSpec(s, lambda i: (0,) * len(s))
    return pl.pallas_call(
        _mlp_body,
        grid=(grid,),
        in_specs=[
            pl.BlockSpec((QB2 * K, ATOM_DIMS), lambda i: (i, 0)),
            pl.BlockSpec((QB2 * K, 16), lambda i: (i, 0)),
            pl.BlockSpec((QB2, 16), lambda i: (i, 0)),
            full(N_LAYERS, h, h),
            full(N_LAYERS, h),
            full(N_LAYERS, h, ATOM_DIMS),
            full(N_LAYERS, ATOM_DIMS),
            full(N_LAYERS, ATOM_DIMS),
            full(N_LAYERS, ATOM_DIMS),
        ],
        out_specs=pl.BlockSpec((QB2, ATOM_DIMS), lambda i: (i, 0)),
        out_shape=jax.ShapeDtypeStruct((n, ATOM_DIMS), jnp.float32),
    )(g, ygath, x16, w1, b1, w2, b2, gamma, beta)


def _run(x, y, y_atom_types, W1, b1, W2, b2, gamma, beta):
    n = x.shape[0]
    vocab, d = y_atom_types.shape
    idx = _knn(x, y)
    y16 = jnp.pad(y, ((0, 0), (0, 16 - y.shape[1])))
    x16 = jnp.pad(x, ((0, 0), (0, 16 - x.shape[1])))
    gather = _make_gather(vocab, d, n * K)
    g, ygath = gather(y_atom_types, idx.reshape(-1), y16)
    return _mlp(g, ygath, x16, W1, b1, W2, b2, gamma, beta)


def kernel(x, y, y_atom_types, W1, b1, W2, b2, gamma, beta):
    devs = jax.devices()
    nd = len(devs)
    n = x.shape[0]
    if nd > 1 and n % (nd * QB2 * 4) == 0:
        mesh = jax.make_mesh((nd,), ("q",))
        p = jax.sharding.PartitionSpec
        ns = lambda spec: jax.sharding.NamedSharding(mesh, spec)
        rep = p()
        args = [jax.reshard(a, ns(s)) for a, s in zip(
            (x, y, y_atom_types, W1, b1, W2, b2, gamma, beta),
            (p("q"), rep, rep, rep, rep, rep, rep, rep, rep))]
        fn = jax.shard_map(
            _run, mesh=mesh,
            in_specs=(p("q"), rep, rep, rep, rep, rep, rep, rep, rep),
            out_specs=p("q"), check_vma=False)
        return fn(*args)
    return _run(x, y, y_atom_types, W1, b1, W2, b2, gamma, beta)
